# baseline (speedup 1.0000x reference)
import functools

import jax
import jax.numpy as jnp
from jax import lax
from jax.experimental import pallas as pl
from jax.experimental.pallas import tpu as pltpu
from jax.experimental.pallas import tpu_sc as plsc

N = 10000
E = 320000
F_IN = 128
HID = 64
NCLS = 32

NW = 32
K = 128
E_TOT = E + N
NB = -(-E_TOT // (NW * K))
E_PAD = NW * NB * K
NPAD = 10240
ROWS_PER_TILE = NPAD // 16

_mesh = plsc.VectorSubcoreMesh(
    core_axis_name="c", subcore_axis_name="s", num_cores=2, num_subcores=16)


def _make_sc_edge(C):
  @functools.partial(
      pl.kernel,
      out_type=(jax.ShapeDtypeStruct((2, NPAD, C), jnp.float32),
                jax.ShapeDtypeStruct((NW, NPAD), jnp.float32)),
      mesh=_mesh,
      compiler_params=pltpu.CompilerParams(
          needs_layout_passes=False, use_tc_tiling_on_sc=False),
      scratch_types=[
          pltpu.VMEM((NB, K), jnp.int32),
          pltpu.VMEM((NB, K), jnp.int32),
          pltpu.VMEM((2, K, C), jnp.float32),
          pltpu.VMEM((2, K, C), jnp.float32),
          pltpu.VMEM((2, K, C), jnp.float32),
          pltpu.VMEM((K // 16, 16, 17), jnp.float32),
          pltpu.VMEM((NPAD,), jnp.float32),
          pltpu.VMEM((C,), jnp.float32),
          pltpu.VMEM_SHARED((NPAD, C), jnp.float32),
          pltpu.SemaphoreType.DMA,
          pltpu.SemaphoreType.DMA,
          pltpu.SemaphoreType.DMA,
          pltpu.SemaphoreType.DMA,
          pltpu.SemaphoreType.DMA,
          pltpu.SemaphoreType.DMA,
      ],
  )
  def sc_edge(xl_hbm, xr_hbm, src_hbm, dst_hbm, att_hbm, acc_out, den_out,
              srcv, dstv, gl, gr, wbuf, tbuf, den, attv, acc_sh,
              sgl0, sgl1, sgr0, sgr1, ssc0, ssc1):
    cid = lax.axis_index("c")
    sid = lax.axis_index("s")
    wid = sid * 2 + cid
    sgl = (sgl0, sgl1)
    sgr = (sgr0, sgr1)
    ssc = (ssc0, ssc1)

    pltpu.sync_copy(src_hbm.at[wid], srcv)
    pltpu.sync_copy(dst_hbm.at[wid], dstv)
    pltpu.sync_copy(att_hbm, attv)

    zero16 = jnp.zeros((16,), jnp.float32)

    def _zw(r, _):
      for buf in range(2):
        for j in range(C // 16):
          wbuf[buf, r, pl.ds(j * 16, 16)] = zero16
      return 0
    lax.fori_loop(0, K, _zw, 0)

    def _zden(i, _):
      den[pl.ds(i * 16, 16)] = zero16
      return 0
    lax.fori_loop(0, NPAD // 16, _zden, 0)

    row0 = sid * ROWS_PER_TILE
    for i in range(ROWS_PER_TILE // K):
      pltpu.sync_copy(wbuf.at[0], acc_sh.at[pl.ds(row0 + i * K, K)])
    plsc.subcore_barrier()

    lanes = lax.iota(jnp.int32, 16)
    att_c = [attv[pl.ds(q * 16, 16)] for q in range(C // 16)]

    def _issue_gathers(b, buf):
      pltpu.async_copy(xl_hbm.at[srcv.at[b]], gl.at[buf], sgl[buf])
      pltpu.async_copy(xr_hbm.at[dstv.at[b]], gr.at[buf], sgr[buf])

    def _wait_gathers(buf):
      pltpu.make_async_copy(xl_hbm.at[srcv.at[0]], gl.at[buf], sgl[buf]).wait()
      pltpu.make_async_copy(xr_hbm.at[dstv.at[0]], gr.at[buf], sgr[buf]).wait()

    def _wait_scatter(buf):
      pltpu.make_async_copy(
          wbuf.at[buf], acc_sh.at[dstv.at[0]], ssc[buf]).wait()

    def _compute_batch(b, buf):
      glb = gl.at[buf]
      grb = gr.at[buf]
      wb = wbuf.at[buf]

      _wait_scatter(buf)

      nq = C // 16

      @plsc.parallel_loop(0, K // 16, unroll=2)
      def _grp(g):
        row0g = g * 16
        tb = tbuf.at[g]
        ts = []
        for e in range(16):
          row = row0g + e
          t = None
          for q in range(nq):
            s = glb[row, pl.ds(q * 16, 16)] + grb[row, pl.ds(q * 16, 16)]
            s = jnp.maximum(s, s * 0.2)
            sq = s * att_c[q]
            t = sq if t is None else t + sq
          ts.append(t)
        for e in range(16):
          tb[e, pl.ds(0, 16)] = ts[e]
        cols = [
            plsc.load_gather(tb, [lanes, jnp.full((16,), c, jnp.int32)])
            for c in range(16)
        ]
        while len(cols) > 1:
          cols = [cols[i] + cols[i + 1] for i in range(0, len(cols), 2)]
        pv = jnp.exp(cols[0])
        dvals = dstv[b, pl.ds(row0g, 16)]
        plsc.addupdate_scatter(den, [dvals], pv)
        for blk in range(0, 16, 4):
          vals = {}
          for e in range(blk, blk + 4):
            row = row0g + e
            for q in range(nq):
              vals[(e, q)] = glb[row, pl.ds(q * 16, 16)] * pv[e]
          for e in range(blk, blk + 4):
            row = row0g + e
            for q in range(nq):
              wb[row, pl.ds(q * 16, 16)] = vals[(e, q)]
      pltpu.async_copy(wb, acc_sh.at[dstv.at[b]], ssc[buf], add=True)

    pltpu.async_copy(wbuf.at[0], acc_sh.at[srcv.at[0]], ssc0, add=True)
    pltpu.async_copy(wbuf.at[1], acc_sh.at[srcv.at[0]], ssc1, add=True)

    _issue_gathers(0, 0)

    def _pair(i, _):
      a = 2 * i
      _issue_gathers(a + 1, 1)
      _wait_gathers(0)
      _compute_batch(a, 0)
      _issue_gathers(a + 2, 0)
      _wait_gathers(1)
      _compute_batch(a + 1, 1)
      return 0

    assert NB % 2 == 1
    lax.fori_loop(0, (NB - 1) // 2, _pair, 0)

    _wait_gathers(0)
    _compute_batch(NB - 1, 0)
    _wait_scatter(0)
    _wait_scatter(1)

    plsc.subcore_barrier()
    pltpu.sync_copy(den, den_out.at[wid])
    pltpu.sync_copy(acc_sh.at[pl.ds(row0, ROWS_PER_TILE)],
                    acc_out.at[cid, pl.ds(row0, ROWS_PER_TILE)])

  return sc_edge


_sc_edge64 = _make_sc_edge(HID)
_sc_edge32 = _make_sc_edge(NCLS)


def _mm1_body(x_ref, wl_ref, wr_ref, xl_ref, xr_ref):
  i = pl.program_id(0)
  rows = i * 1024 + lax.broadcasted_iota(jnp.int32, (1024, 1), 0)
  xb = jnp.where(rows < N, x_ref[...], 0.0)
  xl_ref[...] = jnp.dot(xb, wl_ref[...], preferred_element_type=jnp.float32)
  xr_ref[...] = jnp.dot(xb, wr_ref[...], preferred_element_type=jnp.float32)


def _mm1(x, Wl1, Wr1):
  B = 1024
  return pl.pallas_call(
      _mm1_body,
      grid=(NPAD // B,),
      in_specs=[
          pl.BlockSpec((B, F_IN), lambda i: (i, 0)),
          pl.BlockSpec((F_IN, HID), lambda i: (0, 0)),
          pl.BlockSpec((F_IN, HID), lambda i: (0, 0)),
      ],
      out_specs=[
          pl.BlockSpec((B, HID), lambda i: (i, 0)),
          pl.BlockSpec((B, HID), lambda i: (i, 0)),
      ],
      out_shape=[
          jax.ShapeDtypeStruct((NPAD, HID), jnp.float32),
          jax.ShapeDtypeStruct((NPAD, HID), jnp.float32),
      ],
  )(x, Wl1, Wr1)


def _comb_mm(acc, den, b1, Wl2, Wr2):
  B = 1024
  return pl.pallas_call(
      _comb_mm_body,
      grid=(NPAD // B,),
      in_specs=[
          pl.BlockSpec((1, B, HID), lambda i: (0, i, 0)),
          pl.BlockSpec((1, B, HID), lambda i: (1, i, 0)),
          pl.BlockSpec((NW, B), lambda i: (0, i)),
          pl.BlockSpec((1, HID), lambda i: (0, 0)),
          pl.BlockSpec((HID, NCLS), lambda i: (0, 0)),
          pl.BlockSpec((HID, NCLS), lambda i: (0, 0)),
      ],
      out_specs=[
          pl.BlockSpec((B, NCLS), lambda i: (i, 0)),
          pl.BlockSpec((B, NCLS), lambda i: (i, 0)),
      ],
      out_shape=[
          jax.ShapeDtypeStruct((NPAD, NCLS), jnp.float32),
          jax.ShapeDtypeStruct((NPAD, NCLS), jnp.float32),
      ],
  )(acc, acc, den, b1, Wl2, Wr2)


def _final(acc, den, b2):
  B = 1024
  return pl.pallas_call(
      _final_body,
      grid=(NPAD // B,),
      in_specs=[
          pl.BlockSpec((1, B, NCLS), lambda i: (0, i, 0)),
          pl.BlockSpec((1, B, NCLS), lambda i: (1, i, 0)),
          pl.BlockSpec((NW, B), lambda i: (0, i)),
          pl.BlockSpec((1, NCLS), lambda i: (0, 0)),
      ],
      out_specs=pl.BlockSpec((B, NCLS), lambda i: (i, 0)),
      out_shape=jax.ShapeDtypeStruct((NPAD, NCLS), jnp.float32),
  )(acc, acc, den, b2)


def _comb_mm_body(acc0_ref, acc1_ref, den_ref, b_ref, wl_ref, wr_ref,
                  xl_ref, xr_ref):
  d = jnp.sum(den_ref[...], axis=0)
  h = (acc0_ref[0] + acc1_ref[0]) / (d[:, None] + 1e-16) + b_ref[...]
  h = jnp.maximum(h, 0.0)
  xl_ref[...] = jnp.dot(h, wl_ref[...], preferred_element_type=jnp.float32)
  xr_ref[...] = jnp.dot(h, wr_ref[...], preferred_element_type=jnp.float32)


def _final_body(acc0_ref, acc1_ref, den_ref, b_ref, out_ref):
  d = jnp.sum(den_ref[...], axis=0)
  out_ref[...] = (acc0_ref[0] + acc1_ref[0]) / (d[:, None] + 1e-16) \
      + b_ref[...]


def kernel(x, edge_index, Wl1, Wr1, att1, b1, Wl2, Wr2, att2, b2):
  loops = jnp.arange(N, dtype=jnp.int32)
  pad_e = E_PAD - E_TOT
  src = jnp.concatenate(
      [edge_index[0].astype(jnp.int32), loops,
       jnp.zeros((pad_e,), jnp.int32)]).reshape(NW, NB, K)
  dst = jnp.concatenate(
      [edge_index[1].astype(jnp.int32), loops,
       jnp.full((pad_e,), N, jnp.int32)]).reshape(NW, NB, K)

  xl1, xr1 = _mm1(x, Wl1, Wr1)
  acc1, den1 = _sc_edge64(xl1, xr1, src, dst, att1.reshape(HID))
  xl2, xr2 = _comb_mm(acc1, den1, b1.reshape(1, HID), Wl2, Wr2)
  acc2, den2 = _sc_edge32(xl2, xr2, src, dst, att2.reshape(NCLS))
  return _final(acc2, den2, b2.reshape(1, NCLS))[:N]


# speedup vs baseline: 1.0161x; 1.0161x over previous
import functools

import jax
import jax.numpy as jnp
from jax import lax
from jax.experimental import pallas as pl
from jax.experimental.pallas import tpu as pltpu
from jax.experimental.pallas import tpu_sc as plsc

N = 10000
E = 320000
F_IN = 128
HID = 64
NCLS = 32

NW = 32
K = 128
E_TOT = E + N
NB = -(-E_TOT // (NW * K))
E_PAD = NW * NB * K
NPAD = 10240
ROWS_PER_TILE = NPAD // 16

_mesh = plsc.VectorSubcoreMesh(
    core_axis_name="c", subcore_axis_name="s", num_cores=2, num_subcores=16)


def _make_sc_edge(C):
  CW = C + 16

  @functools.partial(
      pl.kernel,
      out_type=jax.ShapeDtypeStruct((2, NPAD, CW), jnp.float32),
      mesh=_mesh,
      compiler_params=pltpu.CompilerParams(
          needs_layout_passes=False, use_tc_tiling_on_sc=False),
      scratch_types=[
          pltpu.VMEM((NB, K), jnp.int32),
          pltpu.VMEM((NB, K), jnp.int32),
          pltpu.VMEM((2, K, C), jnp.float32),
          pltpu.VMEM((2, K, C), jnp.float32),
          pltpu.VMEM((2, K, CW), jnp.float32),
          pltpu.VMEM((K // 16, 16, 17), jnp.float32),
          pltpu.VMEM((C,), jnp.float32),
          pltpu.VMEM_SHARED((NPAD, CW), jnp.float32),
          pltpu.SemaphoreType.DMA,
          pltpu.SemaphoreType.DMA,
          pltpu.SemaphoreType.DMA,
          pltpu.SemaphoreType.DMA,
          pltpu.SemaphoreType.DMA,
          pltpu.SemaphoreType.DMA,
      ],
  )
  def sc_edge(xl_hbm, xr_hbm, src_hbm, dst_hbm, att_hbm, acc_out,
              srcv, dstv, gl, gr, wbuf, tbuf, attv, acc_sh,
              sgl0, sgl1, sgr0, sgr1, ssc0, ssc1):
    cid = lax.axis_index("c")
    sid = lax.axis_index("s")
    wid = sid * 2 + cid
    sgl = (sgl0, sgl1)
    sgr = (sgr0, sgr1)
    ssc = (ssc0, ssc1)

    pltpu.sync_copy(src_hbm.at[wid], srcv)
    pltpu.sync_copy(dst_hbm.at[wid], dstv)
    pltpu.sync_copy(att_hbm, attv)

    zero16 = jnp.zeros((16,), jnp.float32)

    def _zw(r, _):
      for buf in range(2):
        for j in range(CW // 16):
          wbuf[buf, r, pl.ds(j * 16, 16)] = zero16
      return 0
    lax.fori_loop(0, K, _zw, 0)

    row0 = sid * ROWS_PER_TILE
    for i in range(ROWS_PER_TILE // K):
      pltpu.sync_copy(wbuf.at[0], acc_sh.at[pl.ds(row0 + i * K, K)])
    plsc.subcore_barrier()

    lanes = lax.iota(jnp.int32, 16)
    onehot0 = jnp.where(lanes == 0, 1.0, 0.0).astype(jnp.float32)
    att_c = [attv[pl.ds(q * 16, 16)] for q in range(C // 16)]

    def _issue_gathers(b, buf):
      pltpu.async_copy(xl_hbm.at[srcv.at[b]], gl.at[buf], sgl[buf])
      pltpu.async_copy(xr_hbm.at[dstv.at[b]], gr.at[buf], sgr[buf])

    def _wait_gathers(buf):
      pltpu.make_async_copy(xl_hbm.at[srcv.at[0]], gl.at[buf], sgl[buf]).wait()
      pltpu.make_async_copy(xr_hbm.at[dstv.at[0]], gr.at[buf], sgr[buf]).wait()

    def _wait_scatter(buf):
      pltpu.make_async_copy(
          wbuf.at[buf], acc_sh.at[dstv.at[0]], ssc[buf]).wait()

    def _compute_batch(b, buf):
      glb = gl.at[buf]
      grb = gr.at[buf]
      wb = wbuf.at[buf]

      _wait_scatter(buf)

      nq = C // 16

      @plsc.parallel_loop(0, K // 16, unroll=2)
      def _grp(g):
        row0g = g * 16
        tb = tbuf.at[g]
        ts = []
        for e in range(16):
          row = row0g + e
          t = None
          for q in range(nq):
            s = glb[row, pl.ds(q * 16, 16)] + grb[row, pl.ds(q * 16, 16)]
            s = jnp.maximum(s, s * 0.2)
            sq = s * att_c[q]
            t = sq if t is None else t + sq
          ts.append(t)
        for e in range(16):
          tb[e, pl.ds(0, 16)] = ts[e]
        cols = [
            plsc.load_gather(tb, [lanes, jnp.full((16,), c, jnp.int32)])
            for c in range(16)
        ]
        while len(cols) > 1:
          cols = [cols[i] + cols[i + 1] for i in range(0, len(cols), 2)]
        pv = jnp.exp(cols[0])
        for blk in range(0, 16, 4):
          vals = {}
          for e in range(blk, blk + 4):
            row = row0g + e
            for q in range(nq):
              vals[(e, q)] = glb[row, pl.ds(q * 16, 16)] * pv[e]
          for e in range(blk, blk + 4):
            row = row0g + e
            for q in range(nq):
              wb[row, pl.ds(q * 16, 16)] = vals[(e, q)]
            wb[row, pl.ds(C, 16)] = pv[e] * onehot0
      pltpu.async_copy(wb, acc_sh.at[dstv.at[b]], ssc[buf], add=True)

    pltpu.async_copy(wbuf.at[0], acc_sh.at[srcv.at[0]], ssc0, add=True)
    pltpu.async_copy(wbuf.at[1], acc_sh.at[srcv.at[0]], ssc1, add=True)

    _issue_gathers(0, 0)

    def _pair(i, _):
      a = 2 * i
      _issue_gathers(a + 1, 1)
      _wait_gathers(0)
      _compute_batch(a, 0)
      _issue_gathers(a + 2, 0)
      _wait_gathers(1)
      _compute_batch(a + 1, 1)
      return 0

    assert NB % 2 == 1
    lax.fori_loop(0, (NB - 1) // 2, _pair, 0)

    _wait_gathers(0)
    _compute_batch(NB - 1, 0)
    _wait_scatter(0)
    _wait_scatter(1)

    plsc.subcore_barrier()
    pltpu.sync_copy(acc_sh.at[pl.ds(row0, ROWS_PER_TILE)],
                    acc_out.at[cid, pl.ds(row0, ROWS_PER_TILE)])

  return sc_edge


_sc_edge64 = _make_sc_edge(HID)
_sc_edge32 = _make_sc_edge(NCLS)


def _mm1_body(x_ref, wl_ref, wr_ref, xl_ref, xr_ref):
  i = pl.program_id(0)
  rows = i * 1024 + lax.broadcasted_iota(jnp.int32, (1024, 1), 0)
  xb = jnp.where(rows < N, x_ref[...], 0.0)
  xl_ref[...] = jnp.dot(xb, wl_ref[...], preferred_element_type=jnp.float32)
  xr_ref[...] = jnp.dot(xb, wr_ref[...], preferred_element_type=jnp.float32)


def _mm1(x, Wl1, Wr1):
  B = 1024
  return pl.pallas_call(
      _mm1_body,
      grid=(NPAD // B,),
      in_specs=[
          pl.BlockSpec((B, F_IN), lambda i: (i, 0)),
          pl.BlockSpec((F_IN, HID), lambda i: (0, 0)),
          pl.BlockSpec((F_IN, HID), lambda i: (0, 0)),
      ],
      out_specs=[
          pl.BlockSpec((B, HID), lambda i: (i, 0)),
          pl.BlockSpec((B, HID), lambda i: (i, 0)),
      ],
      out_shape=[
          jax.ShapeDtypeStruct((NPAD, HID), jnp.float32),
          jax.ShapeDtypeStruct((NPAD, HID), jnp.float32),
      ],
  )(x, Wl1, Wr1)


def _comb_mm(acc, b1, Wl2, Wr2):
  B = 1024
  CW = HID + 16
  return pl.pallas_call(
      _comb_mm_body,
      grid=(NPAD // B,),
      in_specs=[
          pl.BlockSpec((1, B, CW), lambda i: (0, i, 0)),
          pl.BlockSpec((1, B, CW), lambda i: (1, i, 0)),
          pl.BlockSpec((1, HID), lambda i: (0, 0)),
          pl.BlockSpec((HID, NCLS), lambda i: (0, 0)),
          pl.BlockSpec((HID, NCLS), lambda i: (0, 0)),
      ],
      out_specs=[
          pl.BlockSpec((B, NCLS), lambda i: (i, 0)),
          pl.BlockSpec((B, NCLS), lambda i: (i, 0)),
      ],
      out_shape=[
          jax.ShapeDtypeStruct((NPAD, NCLS), jnp.float32),
          jax.ShapeDtypeStruct((NPAD, NCLS), jnp.float32),
      ],
  )(acc, acc, b1, Wl2, Wr2)


def _final(acc, b2):
  B = 1024
  CW = NCLS + 16
  return pl.pallas_call(
      _final_body,
      grid=(NPAD // B,),
      in_specs=[
          pl.BlockSpec((1, B, CW), lambda i: (0, i, 0)),
          pl.BlockSpec((1, B, CW), lambda i: (1, i, 0)),
          pl.BlockSpec((1, NCLS), lambda i: (0, 0)),
      ],
      out_specs=pl.BlockSpec((B, NCLS), lambda i: (i, 0)),
      out_shape=jax.ShapeDtypeStruct((NPAD, NCLS), jnp.float32),
  )(acc, acc, b2)


def _comb_mm_body(acc0_ref, acc1_ref, b_ref, wl_ref, wr_ref,
                  xl_ref, xr_ref):
  a = acc0_ref[0] + acc1_ref[0]
  h = a[:, :HID] / (a[:, HID:HID + 1] + 1e-16) + b_ref[...]
  h = jnp.maximum(h, 0.0)
  xl_ref[...] = jnp.dot(h, wl_ref[...], preferred_element_type=jnp.float32)
  xr_ref[...] = jnp.dot(h, wr_ref[...], preferred_element_type=jnp.float32)


def _final_body(acc0_ref, acc1_ref, b_ref, out_ref):
  a = acc0_ref[0] + acc1_ref[0]
  out_ref[...] = a[:, :NCLS] / (a[:, NCLS:NCLS + 1] + 1e-16) + b_ref[...]


def kernel(x, edge_index, Wl1, Wr1, att1, b1, Wl2, Wr2, att2, b2):
  loops = jnp.arange(N, dtype=jnp.int32)
  pad_e = E_PAD - E_TOT
  src = jnp.concatenate(
      [edge_index[0].astype(jnp.int32), loops,
       jnp.zeros((pad_e,), jnp.int32)]).reshape(NW, NB, K)
  dst = jnp.concatenate(
      [edge_index[1].astype(jnp.int32), loops,
       jnp.full((pad_e,), N, jnp.int32)]).reshape(NW, NB, K)

  xl1, xr1 = _mm1(x, Wl1, Wr1)
  acc1 = _sc_edge64(xl1, xr1, src, dst, att1.reshape(HID))
  xl2, xr2 = _comb_mm(acc1, b1.reshape(1, HID), Wl2, Wr2)
  acc2 = _sc_edge32(xl2, xr2, src, dst, att2.reshape(NCLS))
  return _final(acc2, b2.reshape(1, NCLS))[:N]


# speedup vs baseline: 1.0190x; 1.0029x over previous
import functools

import jax
import jax.numpy as jnp
from jax import lax
from jax.experimental import pallas as pl
from jax.experimental.pallas import tpu as pltpu
from jax.experimental.pallas import tpu_sc as plsc

N = 10000
E = 320000
F_IN = 128
HID = 64
NCLS = 32

NW = 32
K = 128
E_TOT = E + N
NB = -(-E_TOT // (NW * K))
E_PAD = NW * NB * K
NPAD = 10240
ROWS_PER_TILE = NPAD // 16

_mesh = plsc.VectorSubcoreMesh(
    core_axis_name="c", subcore_axis_name="s", num_cores=2, num_subcores=16)


def _make_sc_edge(C):
  CW = C + 16

  @functools.partial(
      pl.kernel,
      out_type=jax.ShapeDtypeStruct((2, NPAD, CW), jnp.float32),
      mesh=_mesh,
      compiler_params=pltpu.CompilerParams(
          needs_layout_passes=False, use_tc_tiling_on_sc=False),
      scratch_types=[
          pltpu.VMEM((NB, K), jnp.int32),
          pltpu.VMEM((NB, K), jnp.int32),
          pltpu.VMEM((2, K, C), jnp.float32),
          pltpu.VMEM((2, K, C), jnp.float32),
          pltpu.VMEM((2, K, CW), jnp.float32),
          pltpu.VMEM((K // 16, 16, 17), jnp.float32),
          pltpu.VMEM((C,), jnp.float32),
          pltpu.VMEM_SHARED((NPAD, CW), jnp.float32),
          pltpu.SemaphoreType.DMA,
          pltpu.SemaphoreType.DMA,
          pltpu.SemaphoreType.DMA,
          pltpu.SemaphoreType.DMA,
          pltpu.SemaphoreType.DMA,
          pltpu.SemaphoreType.DMA,
      ],
  )
  def sc_edge(xl_hbm, xr_hbm, src_hbm, dst_hbm, att_hbm, acc_out,
              srcv, dstv, gl, gr, wbuf, tbuf, attv, acc_sh,
              sgl0, sgl1, sgr0, sgr1, ssc0, ssc1):
    cid = lax.axis_index("c")
    sid = lax.axis_index("s")
    wid = sid * 2 + cid
    sgl = (sgl0, sgl1)
    sgr = (sgr0, sgr1)
    ssc = (ssc0, ssc1)

    pltpu.sync_copy(src_hbm.at[wid], srcv)
    pltpu.sync_copy(dst_hbm.at[wid], dstv)

    pltpu.async_copy(xl_hbm.at[srcv.at[0]], gl.at[0], sgl0)
    pltpu.async_copy(xr_hbm.at[dstv.at[0]], gr.at[0], sgr0)
    pltpu.sync_copy(att_hbm, attv)

    zero16 = jnp.zeros((16,), jnp.float32)

    def _zw0(r, _):
      for j in range(CW // 16):
        wbuf[0, r, pl.ds(j * 16, 16)] = zero16
      return 0
    lax.fori_loop(0, K, _zw0, 0)

    row0 = sid * ROWS_PER_TILE
    zcopies = [
        pltpu.async_copy(wbuf.at[0], acc_sh.at[pl.ds(row0 + i * K, K)], ssc0)
        for i in range(ROWS_PER_TILE // K)
    ]

    def _zw1(r, _):
      for j in range(CW // 16):
        wbuf[1, r, pl.ds(j * 16, 16)] = zero16
      return 0
    lax.fori_loop(0, K, _zw1, 0)

    for cp in zcopies:
      cp.wait()
    plsc.subcore_barrier()

    lanes = lax.iota(jnp.int32, 16)
    onehot0 = jnp.where(lanes == 0, 1.0, 0.0).astype(jnp.float32)
    att_c = [attv[pl.ds(q * 16, 16)] for q in range(C // 16)]

    def _issue_gathers(b, buf):
      pltpu.async_copy(xl_hbm.at[srcv.at[b]], gl.at[buf], sgl[buf])
      pltpu.async_copy(xr_hbm.at[dstv.at[b]], gr.at[buf], sgr[buf])

    def _wait_gathers(buf):
      pltpu.make_async_copy(xl_hbm.at[srcv.at[0]], gl.at[buf], sgl[buf]).wait()
      pltpu.make_async_copy(xr_hbm.at[dstv.at[0]], gr.at[buf], sgr[buf]).wait()

    def _wait_scatter(buf):
      pltpu.make_async_copy(
          wbuf.at[buf], acc_sh.at[dstv.at[0]], ssc[buf]).wait()

    def _compute_batch(b, buf):
      glb = gl.at[buf]
      grb = gr.at[buf]
      wb = wbuf.at[buf]

      _wait_scatter(buf)

      nq = C // 16

      @plsc.parallel_loop(0, K // 16, unroll=2)
      def _grp(g):
        row0g = g * 16
        tb = tbuf.at[g]
        ts = []
        for e in range(16):
          row = row0g + e
          t = None
          for q in range(nq):
            s = glb[row, pl.ds(q * 16, 16)] + grb[row, pl.ds(q * 16, 16)]
            s = jnp.maximum(s, s * 0.2)
            sq = s * att_c[q]
            t = sq if t is None else t + sq
          ts.append(t)
        for e in range(16):
          tb[e, pl.ds(0, 16)] = ts[e]
        cols = [
            plsc.load_gather(tb, [lanes, jnp.full((16,), c, jnp.int32)])
            for c in range(16)
        ]
        while len(cols) > 1:
          cols = [cols[i] + cols[i + 1] for i in range(0, len(cols), 2)]
        pv = jnp.exp(cols[0])
        for blk in range(0, 16, 4):
          vals = {}
          for e in range(blk, blk + 4):
            row = row0g + e
            for q in range(nq):
              vals[(e, q)] = glb[row, pl.ds(q * 16, 16)] * pv[e]
          for e in range(blk, blk + 4):
            row = row0g + e
            for q in range(nq):
              wb[row, pl.ds(q * 16, 16)] = vals[(e, q)]
            wb[row, pl.ds(C, 16)] = pv[e] * onehot0
      pltpu.async_copy(wb, acc_sh.at[dstv.at[b]], ssc[buf], add=True)

    pltpu.async_copy(wbuf.at[0], acc_sh.at[srcv.at[0]], ssc0, add=True)
    pltpu.async_copy(wbuf.at[1], acc_sh.at[srcv.at[0]], ssc1, add=True)

    def _pair(i, _):
      a = 2 * i
      _issue_gathers(a + 1, 1)
      _wait_gathers(0)
      _compute_batch(a, 0)
      _issue_gathers(a + 2, 0)
      _wait_gathers(1)
      _compute_batch(a + 1, 1)
      return 0

    assert NB % 2 == 1
    lax.fori_loop(0, (NB - 1) // 2, _pair, 0)

    _wait_gathers(0)
    _compute_batch(NB - 1, 0)
    _wait_scatter(0)
    _wait_scatter(1)

    plsc.subcore_barrier()
    pltpu.sync_copy(acc_sh.at[pl.ds(row0, ROWS_PER_TILE)],
                    acc_out.at[cid, pl.ds(row0, ROWS_PER_TILE)])

  return sc_edge


_sc_edge64 = _make_sc_edge(HID)
_sc_edge32 = _make_sc_edge(NCLS)


def _mm1_body(x_ref, wl_ref, wr_ref, xl_ref, xr_ref):
  i = pl.program_id(0)
  rows = i * 1024 + lax.broadcasted_iota(jnp.int32, (1024, 1), 0)
  xb = jnp.where(rows < N, x_ref[...], 0.0)
  xl_ref[...] = jnp.dot(xb, wl_ref[...], preferred_element_type=jnp.float32)
  xr_ref[...] = jnp.dot(xb, wr_ref[...], preferred_element_type=jnp.float32)


def _mm1(x, Wl1, Wr1):
  B = 1024
  return pl.pallas_call(
      _mm1_body,
      grid=(NPAD // B,),
      in_specs=[
          pl.BlockSpec((B, F_IN), lambda i: (i, 0)),
          pl.BlockSpec((F_IN, HID), lambda i: (0, 0)),
          pl.BlockSpec((F_IN, HID), lambda i: (0, 0)),
      ],
      out_specs=[
          pl.BlockSpec((B, HID), lambda i: (i, 0)),
          pl.BlockSpec((B, HID), lambda i: (i, 0)),
      ],
      out_shape=[
          jax.ShapeDtypeStruct((NPAD, HID), jnp.float32),
          jax.ShapeDtypeStruct((NPAD, HID), jnp.float32),
      ],
  )(x, Wl1, Wr1)


def _comb_mm(acc, b1, Wl2, Wr2):
  B = 1024
  CW = HID + 16
  return pl.pallas_call(
      _comb_mm_body,
      grid=(NPAD // B,),
      in_specs=[
          pl.BlockSpec((1, B, CW), lambda i: (0, i, 0)),
          pl.BlockSpec((1, B, CW), lambda i: (1, i, 0)),
          pl.BlockSpec((1, HID), lambda i: (0, 0)),
          pl.BlockSpec((HID, NCLS), lambda i: (0, 0)),
          pl.BlockSpec((HID, NCLS), lambda i: (0, 0)),
      ],
      out_specs=[
          pl.BlockSpec((B, NCLS), lambda i: (i, 0)),
          pl.BlockSpec((B, NCLS), lambda i: (i, 0)),
      ],
      out_shape=[
          jax.ShapeDtypeStruct((NPAD, NCLS), jnp.float32),
          jax.ShapeDtypeStruct((NPAD, NCLS), jnp.float32),
      ],
  )(acc, acc, b1, Wl2, Wr2)


def _final(acc, b2):
  B = 1024
  CW = NCLS + 16
  return pl.pallas_call(
      _final_body,
      grid=(NPAD // B,),
      in_specs=[
          pl.BlockSpec((1, B, CW), lambda i: (0, i, 0)),
          pl.BlockSpec((1, B, CW), lambda i: (1, i, 0)),
          pl.BlockSpec((1, NCLS), lambda i: (0, 0)),
      ],
      out_specs=pl.BlockSpec((B, NCLS), lambda i: (i, 0)),
      out_shape=jax.ShapeDtypeStruct((NPAD, NCLS), jnp.float32),
  )(acc, acc, b2)


def _comb_mm_body(acc0_ref, acc1_ref, b_ref, wl_ref, wr_ref,
                  xl_ref, xr_ref):
  a = acc0_ref[0] + acc1_ref[0]
  h = a[:, :HID] / (a[:, HID:HID + 1] + 1e-16) + b_ref[...]
  h = jnp.maximum(h, 0.0)
  xl_ref[...] = jnp.dot(h, wl_ref[...], preferred_element_type=jnp.float32)
  xr_ref[...] = jnp.dot(h, wr_ref[...], preferred_element_type=jnp.float32)


def _final_body(acc0_ref, acc1_ref, b_ref, out_ref):
  a = acc0_ref[0] + acc1_ref[0]
  out_ref[...] = a[:, :NCLS] / (a[:, NCLS:NCLS + 1] + 1e-16) + b_ref[...]


def kernel(x, edge_index, Wl1, Wr1, att1, b1, Wl2, Wr2, att2, b2):
  loops = jnp.arange(N, dtype=jnp.int32)
  pad_e = E_PAD - E_TOT
  src = jnp.concatenate(
      [edge_index[0].astype(jnp.int32), loops,
       jnp.zeros((pad_e,), jnp.int32)]).reshape(NW, NB, K)
  dst = jnp.concatenate(
      [edge_index[1].astype(jnp.int32), loops,
       jnp.full((pad_e,), N, jnp.int32)]).reshape(NW, NB, K)

  xl1, xr1 = _mm1(x, Wl1, Wr1)
  acc1 = _sc_edge64(xl1, xr1, src, dst, att1.reshape(HID))
  xl2, xr2 = _comb_mm(acc1, b1.reshape(1, HID), Wl2, Wr2)
  acc2 = _sc_edge32(xl2, xr2, src, dst, att2.reshape(NCLS))
  return _final(acc2, b2.reshape(1, NCLS))[:N]
